# Initial kernel scaffold; baseline (speedup 1.0000x reference)
import jax
import jax.numpy as jnp
from jax.experimental import pallas as pl


def kernel(node_type_ids, edge_index, edge_weight, node_emb, type_emb, W1, b1, W2, b2):
    raise NotImplementedError("write your pallas kernel here")


# speedup vs baseline: 4.9230x; 4.9230x over previous
import functools

import jax
import jax.numpy as jnp
from jax import lax
from jax.experimental import pallas as pl
from jax.experimental.pallas import tpu as pltpu
from jax.experimental.pallas import tpu_sc as plsc

N = 10000
E = 320000
H = 128
D = 64
T = 10

NC = 2
NS = 16
NW = NC * NS
L = 16

_MESH = plsc.VectorSubcoreMesh(core_axis_name="c", subcore_axis_name="s")

XCK = 80
X_CHUNKS = N // XCK
X_ITERS = (X_CHUNKS + NW - 1) // NW


def _x_body(tid_hbm, node_hbm, type_hbm, x_hbm, idx_v, trow_v, nrow_v, sem):
    cid = lax.axis_index("c")
    sid = lax.axis_index("s")
    wid = sid * NC + cid

    def chunk(k, carry):
        c = wid + k * NW

        @pl.when(c < X_CHUNKS)
        def _():
            base = c * XCK
            pltpu.sync_copy(tid_hbm.at[pl.ds(base, XCK)], idx_v)
            pltpu.async_copy(type_hbm.at[idx_v], trow_v, sem).wait()
            pltpu.sync_copy(node_hbm.at[pl.ds(base, XCK)], nrow_v)

            def rowloop(i, c2):
                for hs in range(H // L):
                    sl = pl.ds(hs * L, L)
                    nrow_v[i, sl] = nrow_v[i, sl] + trow_v[i, sl]
                return c2

            lax.fori_loop(0, XCK, rowloop, 0)
            pltpu.sync_copy(nrow_v, x_hbm.at[pl.ds(base, XCK)])

        return carry

    lax.fori_loop(0, X_ITERS, chunk, 0)


@jax.jit
def _x_compute(node_type_ids, node_emb, type_emb):
    f = pl.kernel(
        _x_body,
        out_type=jax.ShapeDtypeStruct((N, H), jnp.float32),
        mesh=_MESH,
        scratch_types=[
            pltpu.VMEM((XCK,), jnp.int32),
            pltpu.VMEM((XCK, H), jnp.float32),
            pltpu.VMEM((XCK, H), jnp.float32),
            pltpu.SemaphoreType.DMA,
        ],
    )
    return f(node_type_ids, node_emb, type_emb)


ECK = 128
E_CHUNKS = E // ECK
E_ITERS = (E_CHUNKS + NW - 1) // NW
NPAD = 10240
NPS = NPAD // NS
ZCK = 128


def _spmm_body(edge_hbm, ew_hbm, x_hbm, out_hbm,
               col_v, row_v, ew_v, rows_v, zbuf_v, acc_sh, sem):
    cid = lax.axis_index("c")
    sid = lax.axis_index("s")
    wid = sid * NC + cid

    def zloop(i, carry):
        for hs in range(H // L):
            zbuf_v[i, pl.ds(hs * L, L)] = jnp.zeros((L,), jnp.float32)
        return carry

    lax.fori_loop(0, ZCK, zloop, 0)
    for b in range(NPS // ZCK):
        pltpu.sync_copy(zbuf_v, acc_sh.at[pl.ds(sid * NPS + b * ZCK, ZCK)])
    plsc.subcore_barrier()

    def chunk(k, carry):
        c = wid + k * NW

        @pl.when(c < E_CHUNKS)
        def _():
            base = c * ECK
            pltpu.sync_copy(edge_hbm.at[0, pl.ds(base, ECK)], row_v)
            pltpu.sync_copy(edge_hbm.at[1, pl.ds(base, ECK)], col_v)
            pltpu.sync_copy(ew_hbm.at[pl.ds(base, ECK)], ew_v)
            pltpu.async_copy(x_hbm.at[col_v], rows_v, sem).wait()

            def scale(g, c2):
                w16g = ew_v[pl.ds(g * L, L)]
                for j in range(L):
                    w16 = lax.broadcast_in_dim(w16g[j], (L,), ())
                    e = g * L + j
                    for hs in range(H // L):
                        sl = pl.ds(hs * L, L)
                        rows_v[e, sl] = rows_v[e, sl] * w16
                return c2

            lax.fori_loop(0, ECK // L, scale, 0)
            pltpu.sync_copy(rows_v, acc_sh.at[row_v], add=True)

        return carry

    lax.fori_loop(0, E_ITERS, chunk, 0)
    plsc.subcore_barrier()
    pltpu.sync_copy(acc_sh.at[pl.ds(sid * NPS, NPS)],
                    out_hbm.at[cid, pl.ds(sid * NPS, NPS)])


@jax.jit
def _spmm(edge_index, edge_weight, x):
    f = pl.kernel(
        _spmm_body,
        out_type=jax.ShapeDtypeStruct((NC, NPAD, H), jnp.float32),
        mesh=_MESH,
        scratch_types=[
            pltpu.VMEM((ECK,), jnp.int32),
            pltpu.VMEM((ECK,), jnp.int32),
            pltpu.VMEM((ECK,), jnp.float32),
            pltpu.VMEM((ECK, H), jnp.float32),
            pltpu.VMEM((ZCK, H), jnp.float32),
            pltpu.VMEM_SHARED((NPAD, H), jnp.float32),
            pltpu.SemaphoreType.DMA,
        ],
    )
    return f(edge_index, edge_weight, x)


BR = 2000


def _mm_body(relu, a_ref, w_ref, bias_ref, o_ref):
    s = a_ref[0] + a_ref[1]
    r = jnp.dot(s, w_ref[...], preferred_element_type=jnp.float32) + bias_ref[...]
    if relu:
        r = jnp.maximum(r, 0.0)
    o_ref[...] = r


def _mm(parts, w, bias, relu):
    dout = w.shape[1]
    return pl.pallas_call(
        functools.partial(_mm_body, relu),
        grid=(N // BR,),
        in_specs=[
            pl.BlockSpec((NC, BR, H), lambda i: (0, i, 0)),
            pl.BlockSpec((H, dout), lambda i: (0, 0)),
            pl.BlockSpec((1, dout), lambda i: (0, 0)),
        ],
        out_specs=pl.BlockSpec((BR, dout), lambda i: (i, 0)),
        out_shape=jax.ShapeDtypeStruct((N, dout), jnp.float32),
    )(parts, w, bias)


def kernel(node_type_ids, edge_index, edge_weight, node_emb, type_emb, W1, b1, W2, b2):
    x = _x_compute(node_type_ids, node_emb, type_emb)
    s1 = _spmm(edge_index, edge_weight, x)
    h = _mm(s1, W1, b1.reshape(1, H), relu=True)
    s2 = _spmm(edge_index, edge_weight, h)
    z = _mm(s2, W2, b2.reshape(1, D), relu=False)
    return z


# speedup vs baseline: 8.3847x; 1.7032x over previous
import functools

import jax
import jax.numpy as jnp
from jax import lax
from jax.experimental import pallas as pl
from jax.experimental.pallas import tpu as pltpu
from jax.experimental.pallas import tpu_sc as plsc

N = 10000
E = 320000
H = 128
D = 64
T = 10

NC = 2
NS = 16
NW = NC * NS
L = 16

_MESH = plsc.VectorSubcoreMesh(core_axis_name="c", subcore_axis_name="s")

XCK = 80
X_CHUNKS = N // XCK
X_ITERS = (X_CHUNKS + NW - 1) // NW


def _x_body(tid_hbm, node_hbm, type_hbm, x_hbm, idx_v, trow_v, nrow_v, sem):
    cid = lax.axis_index("c")
    sid = lax.axis_index("s")
    wid = sid * NC + cid

    def chunk(k, carry):
        c = wid + k * NW

        @pl.when(c < X_CHUNKS)
        def _():
            base = c * XCK
            pltpu.sync_copy(tid_hbm.at[pl.ds(base, XCK)], idx_v)
            pltpu.async_copy(type_hbm.at[idx_v], trow_v, sem).wait()
            pltpu.sync_copy(node_hbm.at[pl.ds(base, XCK)], nrow_v)

            def rowloop(i, c2):
                for hs in range(H // L):
                    sl = pl.ds(hs * L, L)
                    nrow_v[i, sl] = nrow_v[i, sl] + trow_v[i, sl]
                return c2

            lax.fori_loop(0, XCK, rowloop, 0)
            pltpu.sync_copy(nrow_v, x_hbm.at[pl.ds(base, XCK)])

        return carry

    lax.fori_loop(0, X_ITERS, chunk, 0)


@jax.jit
def _x_compute(node_type_ids, node_emb, type_emb):
    f = pl.kernel(
        _x_body,
        out_type=jax.ShapeDtypeStruct((N, H), jnp.float32),
        mesh=_MESH,
        scratch_types=[
            pltpu.VMEM((XCK,), jnp.int32),
            pltpu.VMEM((XCK, H), jnp.float32),
            pltpu.VMEM((XCK, H), jnp.float32),
            pltpu.SemaphoreType.DMA,
        ],
    )
    return f(node_type_ids, node_emb, type_emb)


ECK = 128
E_CHUNKS = E // ECK
E_ITERS = (E_CHUNKS + NW - 1) // NW
NPAD = 10240
NPS = NPAD // NS
ZCK = 128


def _spmm_body(edge_hbm, ew_hbm, x_hbm, out_hbm,
               rc0_v, rc1_v, ew0_v, ew1_v, rows0_v, rows1_v,
               acc_sh, isem0, isem1, gsem0, gsem1):
    cid = lax.axis_index("c")
    sid = lax.axis_index("s")
    wid = sid * NC + cid

    def zloop(i, carry):
        for hs in range(H // L):
            rows0_v[i, pl.ds(hs * L, L)] = jnp.zeros((L,), jnp.float32)
        return carry

    lax.fori_loop(0, ZCK, zloop, 0)
    for b in range(NPS // ZCK):
        pltpu.sync_copy(rows0_v, acc_sh.at[pl.ds(sid * NPS + b * ZCK, ZCK)])
    plsc.subcore_barrier()

    bufs = ((rc0_v, ew0_v, rows0_v, isem0, gsem0),
            (rc1_v, ew1_v, rows1_v, isem1, gsem1))

    def issue_idx(c, buf):
        rc, ew, _, isem, _ = buf
        base = c * ECK
        pltpu.async_copy(edge_hbm.at[pl.ds(0, 2), pl.ds(base, ECK)], rc, isem)
        pltpu.async_copy(ew_hbm.at[pl.ds(base, ECK)], ew, isem)

    def wait_idx(c, buf):
        rc, ew, _, isem, _ = buf
        base = c * ECK
        pltpu.make_async_copy(edge_hbm.at[pl.ds(0, 2), pl.ds(base, ECK)], rc, isem).wait()
        pltpu.make_async_copy(ew_hbm.at[pl.ds(base, ECK)], ew, isem).wait()

    def issue_gather(buf):
        rc, _, rows, _, gsem = buf
        pltpu.async_copy(x_hbm.at[rc.at[1]], rows, gsem)

    def wait_gather(buf):
        rc, _, rows, _, gsem = buf
        pltpu.make_async_copy(x_hbm.at[rc.at[1]], rows, gsem).wait()

    @pl.when(wid < E_CHUNKS)
    def _():
        issue_idx(wid, bufs[0])
        wait_idx(wid, bufs[0])
        issue_gather(bufs[0])

    def step(k, cur, nxt):
        c_cur = wid + k * NW
        c_nxt = c_cur + NW
        rc, ew, rows = cur[0], cur[1], cur[2]

        @pl.when(c_nxt < E_CHUNKS)
        def _():
            issue_idx(c_nxt, nxt)

        @pl.when(c_cur < E_CHUNKS)
        def _():
            wait_gather(cur)

            def scale(g, c2):
                w16g = ew[pl.ds(g * L, L)]
                for j in range(L):
                    w16 = lax.broadcast_in_dim(w16g[j], (L,), ())
                    e = g * L + j
                    for hs in range(H // L):
                        sl = pl.ds(hs * L, L)
                        rows[e, sl] = rows[e, sl] * w16
                return c2

            lax.fori_loop(0, ECK // L, scale, 0)

        @pl.when(c_nxt < E_CHUNKS)
        def _():
            wait_idx(c_nxt, nxt)
            issue_gather(nxt)

        @pl.when(c_cur < E_CHUNKS)
        def _():
            pltpu.sync_copy(rows, acc_sh.at[rc.at[0]], add=True)

    def pair(k2, carry):
        step(2 * k2, bufs[0], bufs[1])
        step(2 * k2 + 1, bufs[1], bufs[0])
        return carry

    lax.fori_loop(0, (E_ITERS + 1) // 2, pair, 0)
    plsc.subcore_barrier()
    pltpu.sync_copy(acc_sh.at[pl.ds(sid * NPS, NPS)],
                    out_hbm.at[cid, pl.ds(sid * NPS, NPS)])


@jax.jit
def _spmm(edge_index, edge_weight, x):
    f = pl.kernel(
        _spmm_body,
        out_type=jax.ShapeDtypeStruct((NC, NPAD, H), jnp.float32),
        mesh=_MESH,
        scratch_types=[
            pltpu.VMEM((2, ECK), jnp.int32),
            pltpu.VMEM((2, ECK), jnp.int32),
            pltpu.VMEM((ECK,), jnp.float32),
            pltpu.VMEM((ECK,), jnp.float32),
            pltpu.VMEM((ECK, H), jnp.float32),
            pltpu.VMEM((ECK, H), jnp.float32),
            pltpu.VMEM_SHARED((NPAD, H), jnp.float32),
            pltpu.SemaphoreType.DMA,
            pltpu.SemaphoreType.DMA,
            pltpu.SemaphoreType.DMA,
            pltpu.SemaphoreType.DMA,
        ],
    )
    return f(edge_index, edge_weight, x)


BR = 2000


def _mm_body(relu, a_ref, w_ref, bias_ref, o_ref):
    s = a_ref[0] + a_ref[1]
    r = jnp.dot(s, w_ref[...], preferred_element_type=jnp.float32) + bias_ref[...]
    if relu:
        r = jnp.maximum(r, 0.0)
    o_ref[...] = r


def _mm(parts, w, bias, relu):
    dout = w.shape[1]
    return pl.pallas_call(
        functools.partial(_mm_body, relu),
        grid=(N // BR,),
        in_specs=[
            pl.BlockSpec((NC, BR, H), lambda i: (0, i, 0)),
            pl.BlockSpec((H, dout), lambda i: (0, 0)),
            pl.BlockSpec((1, dout), lambda i: (0, 0)),
        ],
        out_specs=pl.BlockSpec((BR, dout), lambda i: (i, 0)),
        out_shape=jax.ShapeDtypeStruct((N, dout), jnp.float32),
    )(parts, w, bias)


def kernel(node_type_ids, edge_index, edge_weight, node_emb, type_emb, W1, b1, W2, b2):
    x = _x_compute(node_type_ids, node_emb, type_emb)
    s1 = _spmm(edge_index, edge_weight, x)
    h = _mm(s1, W1, b1.reshape(1, H), relu=True)
    s2 = _spmm(edge_index, edge_weight, h)
    z = _mm(s2, W2, b2.reshape(1, D), relu=False)
    return z
